# baseline (speedup 1.0000x reference)
import functools

import jax
import jax.numpy as jnp
from jax import lax
from jax.experimental import pallas as pl
from jax.experimental.pallas import tpu as pltpu
from jax.experimental.pallas import tpu_sc as plsc

UNK = 1000
D = 64
B = 4096
T = 200
NW = 32
BB = B // NW


def _make_kernel():
    per_w = BB * T
    mesh = plsc.VectorSubcoreMesh(core_axis_name="c", subcore_axis_name="s")

    @functools.partial(
        pl.kernel,
        mesh=mesh,
        compiler_params=pltpu.CompilerParams(use_tc_tiling_on_sc=False,
                                             needs_layout_passes=False),
        out_type=jax.ShapeDtypeStruct((T, D // 8, NW, 8 * BB), jnp.float32),
        scratch_types=[
            pltpu.VMEM((per_w,), jnp.int32),
            pltpu.VMEM((UNK, D), jnp.float32),
            pltpu.VMEM((BB,), jnp.int32),
            pltpu.VMEM((BB,), jnp.int32),
            pltpu.VMEM((BB,), jnp.int32),
            pltpu.VMEM((BB,), jnp.int32),
            pltpu.VMEM((BB, D), jnp.float32),
            pltpu.VMEM((BB, D), jnp.float32),
            pltpu.VMEM((D // 8, 8 * BB), jnp.float32),
            pltpu.VMEM((D // 8, 8 * BB), jnp.float32),
            pltpu.SemaphoreType.DMA,
            pltpu.SemaphoreType.DMA,
            pltpu.SemaphoreType.DMA,
            pltpu.SemaphoreType.DMA,
        ],
    )
    def body(ids_hbm, glove_hbm, unk_hbm, out_hbm,
             ids_all, unk_v, gix_a, gix_b, sid_a, sid_b,
             rows_a, rows_b, tb_a, tb_b,
             sem_ga, sem_gb, sem_wa, sem_wb):
        wid = lax.axis_index("s") * 2 + lax.axis_index("c")
        lane = lax.iota(jnp.int32, 16)
        lane200 = lane * jnp.full((16,), T, jnp.int32)
        c_unk = jnp.full((16,), UNK, jnp.int32)
        c_zero = jnp.zeros((16,), jnp.int32)
        c_one = jnp.full((16,), 1, jnp.int32)
        c_16 = jnp.full((16,), 16, jnp.int32)
        c_step = jnp.full((16,), 16 * T, jnp.int32)
        gixs = [gix_a, gix_b]
        sids = [sid_a, sid_b]
        rowss = [rows_a, rows_b]
        tbs = [tb_a, tb_b]
        sem_g = [sem_ga, sem_gb]
        sem_w = [sem_wa, sem_wb]

        pltpu.sync_copy(
            ids_hbm.at[pl.ds(pl.multiple_of(wid * per_w, BB), per_w)],
            ids_all)
        pltpu.sync_copy(unk_hbm, unk_v)

        def extract_and_fire(t, p):
            gix_v = gixs[p]
            sid_v = sids[p]
            off = lane200 + jnp.full((16,), t, jnp.int32)
            for k in range(BB // 16):
                ids = plsc.load_gather(ids_all, [off])
                sid_v[pl.ds(k * 16, 16)] = ids
                gix_v[pl.ds(k * 16, 16)] = jnp.where(ids < c_unk, c_zero,
                                                     ids - c_unk)
                off = off + c_step
            pltpu.async_copy(glove_hbm.at[gix_v], rowss[p], sem_g[p])

        def wait_writes(p):
            pltpu.make_async_copy(tbs[p], out_hbm.at[0, :, 0], sem_w[p]).wait()

        def wait_gathers(p):
            pltpu.make_async_copy(glove_hbm.at[pl.ds(0, BB)], rowss[p],
                                  sem_g[p]).wait()

        def transpose_and_write(t, q):
            wait_gathers(q)
            rows_v = rowss[q]
            tb_v = tbs[q]
            sid_v = sids[q]
            rvec = lane
            for k in range(BB // 16):
                ids = sid_v[pl.ds(k * 16, 16)]
                m = ids < c_unk
                rv = rvec

                def dloop(d8, cvec, rv=rv, k=k):
                    for dd in range(8):
                        tb_v[d8, pl.ds(dd * BB + k * 16, 16)] = \
                            plsc.load_gather(rows_v, [rv, cvec])
                        cvec = cvec + c_one
                    return cvec

                lax.fori_loop(0, D // 8, dloop, c_zero)
                nm = jnp.sum(jnp.where(m, c_one, c_zero))

                @pl.when(nm > 0)
                def _fixup(m=m, ids=ids, k=k):
                    uix = jnp.where(m, ids, c_zero)

                    def uloop(d8, uvec):
                        for dd in range(8):
                            sl = pl.ds(dd * BB + k * 16, 16)
                            u = plsc.load_gather(unk_v, [uix, uvec])
                            tb_v[d8, sl] = jnp.where(m, u, tb_v[d8, sl])
                            uvec = uvec + c_one
                        return uvec

                    lax.fori_loop(0, D // 8, uloop, c_zero)

                rvec = rvec + c_16
            for d8 in range(D // 8):
                pltpu.async_copy(tb_v.at[d8], out_hbm.at[t, d8, wid],
                                 sem_w[q])

        def pair(i, carry):
            t = i * 2

            extract_and_fire(t, 0)

            @pl.when(t >= 1)
            def _():
                @pl.when(t >= 3)
                def _():
                    wait_writes(1)
                transpose_and_write(t - 1, 1)

            extract_and_fire(t + 1, 1)

            @pl.when(t >= 2)
            def _():
                wait_writes(0)
            transpose_and_write(t, 0)
            return carry

        lax.fori_loop(0, T // 2, pair, 0)

        wait_writes(1)
        transpose_and_write(T - 1, 1)
        wait_writes(0)
        wait_writes(1)

    return body


def kernel(context, glove_table, unk_table):
    b, t = context.shape
    assert (b, t) == (B, T) and glove_table.shape[1] == D
    flat = context.reshape(b * t)
    out4 = _make_kernel()(flat, glove_table, unk_table)
    out5 = out4.reshape(T, D // 8, NW, 8, BB)
    return out5.transpose((2, 4, 0, 1, 3)).reshape(b, t, D)


# speedup vs baseline: 1.7893x; 1.7893x over previous
import functools

import jax
import jax.numpy as jnp
from jax import lax
from jax.experimental import pallas as pl
from jax.experimental.pallas import tpu as pltpu
from jax.experimental.pallas import tpu_sc as plsc

UNK = 1000
D = 64
SUB = 64
SHIFT = 6
C = 512
NSUB = C // SUB


def _make_kernel(L, NW, per_w):
    nch = per_w // C
    assert nch % 2 == 0 and nch >= 4
    prow = per_w // SUB + 1
    mesh = plsc.VectorSubcoreMesh(core_axis_name="c", subcore_axis_name="s")

    @functools.partial(
        pl.kernel,
        mesh=mesh,
        compiler_params=pltpu.CompilerParams(use_tc_tiling_on_sc=False,
                                             needs_layout_passes=False),
        out_type=jax.ShapeDtypeStruct((L, D), jnp.float32),
        scratch_types=[
            pltpu.VMEM((C,), jnp.int32),
            pltpu.VMEM((C,), jnp.int32),
            pltpu.VMEM((C,), jnp.int32),
            pltpu.VMEM((C,), jnp.int32),
            pltpu.VMEM((C, D), jnp.float32),
            pltpu.VMEM((C, D), jnp.float32),
            pltpu.VMEM((prow, SUB), jnp.int32),
            pltpu.VMEM((prow, SUB), jnp.int32),
            pltpu.VMEM((SUB, D), jnp.float32),
            pltpu.SemaphoreType.DMA,
            pltpu.SemaphoreType.DMA,
            pltpu.SemaphoreType.DMA,
            pltpu.SemaphoreType.DMA,
            pltpu.SemaphoreType.DMA,
            pltpu.SemaphoreType.DMA,
            pltpu.SemaphoreType.DMA,
        ],
    )
    def body(ids_hbm, glove_hbm, unk_hbm, out_hbm,
             idx_a, idx_b, gix_a, gix_b, rows_a, rows_b,
             uid_v, upos_v, ubuf_v,
             sem_ia, sem_ib, sem_ga, sem_gb, sem_oa, sem_ob, sem_u):
        wid = lax.axis_index("s") * 2 + lax.axis_index("c")
        base = pl.multiple_of(wid * per_w, C)
        lane = lax.iota(jnp.int32, 16)
        c_unk = jnp.full((16,), UNK, jnp.int32)
        c_zero = jnp.zeros((16,), jnp.int32)
        c_one = jnp.full((16,), 1, jnp.int32)
        c_7 = jnp.full((16,), SHIFT, jnp.int32)
        c_127 = jnp.full((16,), SUB - 1, jnp.int32)
        idxs = [idx_a, idx_b]
        gixs = [gix_a, gix_b]
        rowss = [rows_a, rows_b]
        sem_i = [sem_ia, sem_ib]
        sem_g = [sem_ga, sem_gb]
        sem_o = [sem_oa, sem_ob]

        def fire_ids(g, p):
            gc = lax.min(g, nch - 1)
            b0 = pl.multiple_of(base + gc * C, C)
            pltpu.async_copy(ids_hbm.at[pl.ds(b0, C)], idxs[p], sem_i[p])

        def wait_ids(p):
            pltpu.make_async_copy(ids_hbm.at[pl.ds(0, C)], idxs[p],
                                  sem_i[p]).wait()

        def clamp_pass(p):
            idx_v = idxs[p]
            gix_v = gixs[p]
            for k in range(C // 16):
                o = k * 16
                ids = idx_v[pl.ds(o, 16)]
                gix_v[pl.ds(o, 16)] = jnp.where(ids < c_unk, c_zero,
                                                ids - c_unk)

        def compact_pass(g, p, ucur_vec):
            b0 = pl.multiple_of(base + g * C, C)
            idx_v = idxs[p]
            for k in range(C // 16):
                o = k * 16
                ids = idx_v[pl.ds(o, 16)]
                m = ids < c_unk
                cnt = plsc.all_reduce_population_count(m)
                mi = jnp.where(m, c_one, c_zero)
                excl = plsc.cumsum(mi) - mi
                tgt = ucur_vec + excl
                row = lax.shift_right_logical(tgt, c_7)
                col = lax.bitwise_and(tgt, c_127)
                pos = jnp.full((16,), b0 + o, jnp.int32) + lane
                plsc.store_scatter(uid_v, [row, col], ids, mask=m)
                plsc.store_scatter(upos_v, [row, col], pos, mask=m)
                ucur_vec = ucur_vec + cnt
            return ucur_vec

        def fire_gathers(p):
            gix_v = gixs[p]
            rows_v = rowss[p]
            for j in range(NSUB):
                pltpu.async_copy(
                    glove_hbm.at[gix_v.at[pl.ds(j * SUB, SUB)]],
                    rows_v.at[pl.ds(j * SUB, SUB)],
                    sem_g[p],
                )

        def wait_gathers(p):
            pltpu.make_async_copy(glove_hbm.at[pl.ds(0, C)], rowss[p],
                                  sem_g[p]).wait()

        def fire_copyout(g, p):
            b0 = pl.multiple_of(base + g * C, C)
            pltpu.async_copy(rowss[p], out_hbm.at[pl.ds(b0, C)], sem_o[p])

        def wait_copyout(p):
            pltpu.make_async_copy(rowss[p], out_hbm.at[pl.ds(0, C)],
                                  sem_o[p]).wait()

        def step(g, p, ucur_vec, w_gather, w_copyout):
            wait_ids(p)
            clamp_pass(p)
            if w_gather:
                wait_gathers(1 - p)
                fire_copyout(g - 1, 1 - p)
            if w_copyout:
                wait_copyout(p)
            fire_gathers(p)
            ucur_vec = compact_pass(g, p, ucur_vec)
            fire_ids(g + 2, p)
            return ucur_vec

        fire_ids(0, 0)
        fire_ids(1, 1)
        ucur_vec = step(0, 0, c_zero, False, False)
        ucur_vec = step(1, 1, ucur_vec, True, False)

        def pair(i, ucur_vec):
            g = i * 2
            ucur_vec = step(g, 0, ucur_vec, True, True)
            ucur_vec = step(g + 1, 1, ucur_vec, True, True)
            return ucur_vec

        ucur_vec = lax.fori_loop(1, nch // 2, pair, ucur_vec)

        wait_gathers(1)
        fire_copyout(nch - 1, 1)
        wait_copyout(0)
        wait_copyout(1)
        wait_ids(0)
        wait_ids(1)

        cur = jnp.max(ucur_vec)

        def fire_block(b, carry):
            pltpu.async_copy(unk_hbm.at[uid_v.at[b]], ubuf_v, sem_u).wait()
            pltpu.async_copy(ubuf_v, out_hbm.at[upos_v.at[b]], sem_u).wait()
            return carry

        nfull = lax.shift_right_logical(cur, SHIFT)
        lax.fori_loop(0, nfull, fire_block, 0)

        rem = lax.bitwise_and(cur, SUB - 1)

        @pl.when(rem > 0)
        def _flush():
            lrow = jnp.full((16,), lax.shift_right_logical(cur - 1, SHIFT),
                            jnp.int32)
            lcol = jnp.full((16,), lax.bitwise_and(cur - 1, SUB - 1),
                            jnp.int32)
            padid = plsc.load_gather(uid_v, [lrow, lcol])
            padpos = plsc.load_gather(upos_v, [lrow, lcol])
            prow_v = jnp.full((16,), nfull, jnp.int32)
            for k in range(SUB // 16):
                offs = jnp.full((16,), k * 16, jnp.int32) + lane
                mm = offs >= jnp.full((16,), rem, jnp.int32)
                plsc.store_scatter(uid_v, [prow_v, offs], padid, mask=mm)
                plsc.store_scatter(upos_v, [prow_v, offs], padpos, mask=mm)
            fire_block(nfull, 0)

    return body


def kernel(context, glove_table, unk_table):
    b, t = context.shape
    L = b * t
    NW = 32
    per_w = L // NW
    assert per_w % C == 0
    flat = context.reshape(L)
    out = _make_kernel(L, NW, per_w)(flat, glove_table, unk_table)
    return out.reshape(b, t, D)
